# Initial kernel scaffold; baseline (speedup 1.0000x reference)
import jax
import jax.numpy as jnp
from jax.experimental import pallas as pl


def kernel(x, segment_num, Wq, bq):
    raise NotImplementedError("write your pallas kernel here")


# speedup vs baseline: 1.4160x; 1.4160x over previous
import functools

import jax
import jax.numpy as jnp
from jax import lax
from jax.experimental import pallas as pl
from jax.experimental.pallas import tpu as pltpu
from jax.experimental.pallas import tpu_sc as plsc

B = 256
D = 1024
DC = D // 16
C = 32
NEG = -1e30


@functools.cache
def _build(N):
    info = plsc.get_sparse_core_info()
    n_cores, n_sub = info.num_cores, info.num_subcores
    n_workers = n_cores * n_sub
    pairs_per_w = (B // 2) // n_workers

    mesh = plsc.VectorSubcoreMesh(core_axis_name="c", subcore_axis_name="s")

    @functools.partial(
        pl.kernel,
        out_type=(
            jax.ShapeDtypeStruct(((B - 1) * D,), jnp.float32),
            jax.ShapeDtypeStruct(((B - 1) * D,), jnp.float32),
        ),
        mesh=mesh,
        compiler_params=pltpu.CompilerParams(needs_layout_passes=False),
        scratch_types=[
            pltpu.VMEM((C * D,), jnp.float32),
            pltpu.VMEM((D,), jnp.float32),
            pltpu.VMEM((B + 16,), jnp.int32),
            pltpu.VMEM((C + 16,), jnp.float32),
            pltpu.VMEM((D,), jnp.float32),
            pltpu.VMEM((D,), jnp.float32),
        ],
    )
    def sc_kernel(x_hbm, sn_hbm, wq_hbm, out_hbm, outseg_hbm,
                  rows_v, wq_v, sn_v, wt_v, a_v, s_v):
        wid = lax.axis_index("s") * n_cores + lax.axis_index("c")
        pltpu.sync_copy(wq_hbm, wq_v)
        pltpu.sync_copy(sn_hbm, sn_v.at[pl.ds(0, B)])

        def do_segment(seg):
            @pl.when(seg > 0)
            def _():
                seg_len = seg
                r0 = (seg * (seg - 1)) // 2

                def zero_body(k, _):
                    sl = pl.ds(k * 16, 16)
                    a_v[sl] = jnp.zeros((16,), jnp.float32)
                    s_v[sl] = jnp.zeros((16,), jnp.float32)
                    return 0
                lax.fori_loop(0, DC, zero_body, 0)

                n_chunks = (seg_len + C - 1) // C
                clamp = jnp.maximum(seg_len - C, 0)

                def chunk_body(c, carry):
                    m, svec = carry
                    start = jnp.minimum(c * C, clamp)
                    pltpu.sync_copy(
                        x_hbm.at[pl.ds((r0 + start) * D, C * D)], rows_v)
                    j_lo = c * C - start
                    j_hi = jnp.minimum((c + 1) * C, seg_len) - start

                    lane = jnp.arange(16, dtype=jnp.int32)
                    negs = jnp.full((16,), NEG, jnp.float32)

                    def logit_body(j, carry):
                        lg0, lg1 = carry
                        def dot_body(k, acc):
                            sl = pl.ds(k * 16, 16)
                            return acc + rows_v[pl.ds(j * D + k * 16, 16)] * wq_v[sl]
                        acc = lax.fori_loop(0, DC, dot_body,
                                            jnp.zeros((16,), jnp.float32))
                        bval = jnp.full((16,), jnp.sum(acc), jnp.float32)
                        lg0 = jnp.where(lane == j, bval, lg0)
                        lg1 = jnp.where(lane == j - 16, bval, lg1)
                        return lg0, lg1
                    l0, l1 = lax.fori_loop(j_lo, j_hi, logit_body, (negs, negs))
                    cmax = jnp.maximum(jnp.max(l0), jnp.max(l1))
                    m_new = jnp.maximum(m, cmax)
                    scale = jnp.exp(jnp.full((16,), m - m_new, jnp.float32))
                    e0 = jnp.exp(l0 - m_new)
                    e1 = jnp.exp(l1 - m_new)
                    wt_v[pl.ds(0, 16)] = e0
                    wt_v[pl.ds(16, 16)] = e1
                    svec_new = svec * scale + (e0 + e1)

                    @pl.when(cmax > m)
                    def _rescale():
                        def rs(k, _):
                            sl = pl.ds(k * 16, 16)
                            a_v[sl] = a_v[sl] * scale
                            return 0
                        lax.fori_loop(0, DC, rs, 0)

                    def acc_k(k, _):
                        sl = pl.ds(k * 16, 16)
                        def acc_j(j, carry2):
                            acc_a, acc_s = carry2
                            rc = rows_v[pl.ds(j * D + k * 16, 16)]
                            wj = wt_v[pl.ds(j, 16)][0]
                            return acc_a + rc * wj, acc_s + rc
                        acc_a, acc_s = lax.fori_loop(
                            j_lo, j_hi, acc_j, (a_v[sl], s_v[sl]))
                        a_v[sl] = acc_a
                        s_v[sl] = acc_s
                        return 0
                    lax.fori_loop(0, DC, acc_k, 0)
                    return m_new, svec_new

                m, svec = lax.fori_loop(
                    0, n_chunks, chunk_body,
                    (jnp.float32(NEG), jnp.zeros((16,), jnp.float32)))

                ones = jnp.ones((16,), jnp.float32)
                inv_a = ones / jnp.full((16,), jnp.sum(svec), jnp.float32)
                cnt = sn_v[pl.ds(seg, 16)][0].astype(jnp.float32)
                inv_s = ones / jnp.maximum(
                    jnp.full((16,), cnt, jnp.float32), 1.0)

                def norm_body(k, _):
                    sl = pl.ds(k * 16, 16)
                    a_v[sl] = a_v[sl] * inv_a
                    s_v[sl] = s_v[sl] * inv_s
                    return 0
                lax.fori_loop(0, DC, norm_body, 0)

                pltpu.sync_copy(a_v, out_hbm.at[pl.ds((seg - 1) * D, D)])
                pltpu.sync_copy(s_v, outseg_hbm.at[pl.ds((seg - 1) * D, D)])

        for q in range(pairs_per_w):
            p = wid * pairs_per_w + q
            do_segment(p)
            do_segment(B - 1 - p)

    return sc_kernel


def kernel(x, segment_num, Wq, bq):
    del bq
    out, out_segment = _build(x.shape[0])(
        x.reshape(-1), segment_num, Wq)
    return out.reshape(B - 1, D), out_segment.reshape(B - 1, D)


# speedup vs baseline: 1.6489x; 1.1645x over previous
import functools

import jax
import jax.numpy as jnp
from jax import lax
from jax.experimental import pallas as pl
from jax.experimental.pallas import tpu as pltpu
from jax.experimental.pallas import tpu_sc as plsc

B = 256
D = 1024
DC = D // 16
C = 32
NEG = -1e30


@functools.cache
def _build(N):
    info = plsc.get_sparse_core_info()
    n_cores, n_sub = info.num_cores, info.num_subcores
    n_workers = n_cores * n_sub
    pairs_per_w = (B // 2) // n_workers

    mesh = plsc.VectorSubcoreMesh(core_axis_name="c", subcore_axis_name="s")

    @functools.partial(
        pl.kernel,
        out_type=(
            jax.ShapeDtypeStruct(((B - 1) * D,), jnp.float32),
            jax.ShapeDtypeStruct(((B - 1) * D,), jnp.float32),
        ),
        mesh=mesh,
        compiler_params=pltpu.CompilerParams(needs_layout_passes=False),
        scratch_types=[
            pltpu.VMEM((C * D,), jnp.float32),
            pltpu.VMEM((D,), jnp.float32),
            pltpu.VMEM((B + 16,), jnp.int32),
            pltpu.VMEM((C + 16,), jnp.float32),
            pltpu.VMEM((D,), jnp.float32),
            pltpu.VMEM((D,), jnp.float32),
        ],
    )
    def sc_kernel(x_hbm, sn_hbm, wq_hbm, out_hbm, outseg_hbm,
                  rows_v, wq_v, sn_v, wt_v, a_v, s_v):
        wid = lax.axis_index("s") * n_cores + lax.axis_index("c")
        pltpu.sync_copy(wq_hbm, wq_v)
        pltpu.sync_copy(sn_hbm, sn_v.at[pl.ds(0, B)])

        def do_segment(seg):
            @pl.when(seg > 0)
            def _():
                seg_len = seg
                r0 = (seg * (seg - 1)) // 2

                def zero_body(k, _):
                    sl = pl.ds(k * 16, 16)
                    a_v[sl] = jnp.zeros((16,), jnp.float32)
                    s_v[sl] = jnp.zeros((16,), jnp.float32)
                    return 0
                lax.fori_loop(0, DC, zero_body, 0, unroll=8)

                n_chunks = (seg_len + C - 1) // C
                clamp = jnp.maximum(seg_len - C, 0)

                def chunk_body(c, carry):
                    m, svec = carry
                    start = jnp.minimum(c * C, clamp)
                    pltpu.sync_copy(
                        x_hbm.at[pl.ds((r0 + start) * D, C * D)], rows_v)
                    j_lo = c * C - start
                    j_hi = jnp.minimum((c + 1) * C, seg_len) - start

                    lane = jnp.arange(16, dtype=jnp.int32)
                    negs = jnp.full((16,), NEG, jnp.float32)

                    def logit_body(j, carry):
                        lg0, lg1 = carry
                        def dot_body(k, acc):
                            sl = pl.ds(k * 16, 16)
                            return acc + rows_v[pl.ds(j * D + k * 16, 16)] * wq_v[sl]
                        acc = lax.fori_loop(0, DC, dot_body,
                                            jnp.zeros((16,), jnp.float32),
                                            unroll=8)
                        bval = jnp.full((16,), jnp.sum(acc), jnp.float32)
                        lg0 = jnp.where(lane == j, bval, lg0)
                        lg1 = jnp.where(lane == j - 16, bval, lg1)
                        return lg0, lg1
                    l0, l1 = lax.fori_loop(j_lo, j_hi, logit_body, (negs, negs))
                    cmax = jnp.maximum(jnp.max(l0), jnp.max(l1))
                    m_new = jnp.maximum(m, cmax)
                    scale = jnp.exp(jnp.full((16,), m - m_new, jnp.float32))
                    e0 = jnp.exp(l0 - m_new)
                    e1 = jnp.exp(l1 - m_new)
                    wt_v[pl.ds(0, 16)] = e0
                    wt_v[pl.ds(16, 16)] = e1
                    svec_new = svec * scale + (e0 + e1)

                    @pl.when(cmax > m)
                    def _rescale():
                        def rs(k, _):
                            sl = pl.ds(k * 16, 16)
                            a_v[sl] = a_v[sl] * scale
                            return 0
                        lax.fori_loop(0, DC, rs, 0, unroll=8)

                    def acc_j(j, _):
                        wj = wt_v[pl.ds(j, 16)][0]
                        def acc_k(k, _2):
                            sl = pl.ds(k * 16, 16)
                            rc = rows_v[pl.ds(j * D + k * 16, 16)]
                            plsc.addupdate(a_v.at[sl], rc * wj)
                            plsc.addupdate(s_v.at[sl], rc)
                            return 0
                        lax.fori_loop(0, DC, acc_k, 0, unroll=8)
                        return 0
                    lax.fori_loop(j_lo, j_hi, acc_j, 0)
                    return m_new, svec_new

                m, svec = lax.fori_loop(
                    0, n_chunks, chunk_body,
                    (jnp.float32(NEG), jnp.zeros((16,), jnp.float32)))

                ones = jnp.ones((16,), jnp.float32)
                inv_a = ones / jnp.full((16,), jnp.sum(svec), jnp.float32)
                cnt = sn_v[pl.ds(seg, 16)][0].astype(jnp.float32)
                inv_s = ones / jnp.maximum(
                    jnp.full((16,), cnt, jnp.float32), 1.0)

                def norm_body(k, _):
                    sl = pl.ds(k * 16, 16)
                    a_v[sl] = a_v[sl] * inv_a
                    s_v[sl] = s_v[sl] * inv_s
                    return 0
                lax.fori_loop(0, DC, norm_body, 0, unroll=8)

                pltpu.sync_copy(a_v, out_hbm.at[pl.ds((seg - 1) * D, D)])
                pltpu.sync_copy(s_v, outseg_hbm.at[pl.ds((seg - 1) * D, D)])

        for q in range(pairs_per_w):
            p = wid * pairs_per_w + q
            do_segment(p)
            do_segment(B - 1 - p)

    return sc_kernel


def kernel(x, segment_num, Wq, bq):
    del bq
    out, out_segment = _build(x.shape[0])(
        x.reshape(-1), segment_num, Wq)
    return out.reshape(B - 1, D), out_segment.reshape(B - 1, D)
